# Initial kernel scaffold; baseline (speedup 1.0000x reference)
import jax
import jax.numpy as jnp
from jax.experimental import pallas as pl


def kernel(x, W, b):
    raise NotImplementedError("write your pallas kernel here")


# speedup vs baseline: 6.1201x; 6.1201x over previous
import functools

import jax
import jax.numpy as jnp
from jax.experimental import pallas as pl

TOP_K = 8
ROW_BLOCK = 512


def _gating_body(x_ref, w_ref, b_ref, gates_ref, idx_ref):
    logits = (
        jnp.dot(x_ref[...], w_ref[...], preferred_element_type=jnp.float32)
        + b_ref[...]
    )
    e = logits.shape[-1]
    iota = jax.lax.broadcasted_iota(jnp.int32, logits.shape, 1)
    work = logits
    mask = jnp.zeros(logits.shape, jnp.bool_)
    idx_cols = []
    for _ in range(TOP_K):
        m = jnp.max(work, axis=1, keepdims=True)
        sel_idx = jnp.min(jnp.where(work == m, iota, e), axis=1, keepdims=True)
        idx_cols.append(sel_idx)
        sel = iota == sel_idx
        mask = mask | sel
        work = jnp.where(sel, -jnp.inf, work)
    top1 = jnp.max(logits, axis=1, keepdims=True)
    ex = jnp.where(mask, jnp.exp(logits - top1), 0.0)
    gates_ref[...] = ex / jnp.sum(ex, axis=1, keepdims=True)
    idx_ref[...] = jnp.concatenate(idx_cols, axis=1)


def kernel(x, W, b):
    n_tok, d = x.shape
    e = W.shape[1]
    b2 = b.reshape(1, e)
    grid = (n_tok // ROW_BLOCK,)
    gates, idx = pl.pallas_call(
        _gating_body,
        grid=grid,
        in_specs=[
            pl.BlockSpec((ROW_BLOCK, d), lambda i: (i, 0)),
            pl.BlockSpec((d, e), lambda i: (0, 0)),
            pl.BlockSpec((1, e), lambda i: (0, 0)),
        ],
        out_specs=[
            pl.BlockSpec((ROW_BLOCK, e), lambda i: (i, 0)),
            pl.BlockSpec((ROW_BLOCK, TOP_K), lambda i: (i, 0)),
        ],
        out_shape=[
            jax.ShapeDtypeStruct((n_tok, e), jnp.float32),
            jax.ShapeDtypeStruct((n_tok, TOP_K), jnp.int32),
        ],
    )(x, W, b2)
    return (gates, idx)


# speedup vs baseline: 9.8964x; 1.6170x over previous
import functools

import jax
import jax.numpy as jnp
from jax.experimental import pallas as pl

TOP_K = 8
ROW_BLOCK = 512


def _gating_body(x_ref, w_ref, b_ref, gates_ref, idx_ref):
    logits = (
        jnp.dot(x_ref[...], w_ref[...], preferred_element_type=jnp.float32)
        + b_ref[...]
    )
    iota = jax.lax.broadcasted_iota(jnp.int32, logits.shape, 1)
    work = logits
    mask = jnp.zeros(logits.shape, jnp.bool_)
    idx_cols = []
    for _ in range(TOP_K):
        sel_idx = jnp.argmax(work, axis=1, keepdims=True)
        idx_cols.append(sel_idx)
        sel = iota == sel_idx
        mask = mask | sel
        work = jnp.where(sel, -jnp.inf, work)
    top1 = jnp.max(logits, axis=1, keepdims=True)
    ex = jnp.where(mask, jnp.exp(logits - top1), 0.0)
    gates_ref[...] = ex / jnp.sum(ex, axis=1, keepdims=True)
    idx_ref[...] = jnp.concatenate(idx_cols, axis=1)


def kernel(x, W, b):
    n_tok, d = x.shape
    e = W.shape[1]
    b2 = b.reshape(1, e)
    grid = (n_tok // ROW_BLOCK,)
    gates, idx = pl.pallas_call(
        _gating_body,
        grid=grid,
        in_specs=[
            pl.BlockSpec((ROW_BLOCK, d), lambda i: (i, 0)),
            pl.BlockSpec((d, e), lambda i: (0, 0)),
            pl.BlockSpec((1, e), lambda i: (0, 0)),
        ],
        out_specs=[
            pl.BlockSpec((ROW_BLOCK, e), lambda i: (i, 0)),
            pl.BlockSpec((ROW_BLOCK, TOP_K), lambda i: (i, 0)),
        ],
        out_shape=[
            jax.ShapeDtypeStruct((n_tok, e), jnp.float32),
            jax.ShapeDtypeStruct((n_tok, TOP_K), jnp.int32),
        ],
    )(x, W, b2)
    return (gates, idx)


# speedup vs baseline: 12.2384x; 1.2367x over previous
import functools

import jax
import jax.numpy as jnp
from jax.experimental import pallas as pl
from jax.experimental.pallas import tpu as pltpu

TOP_K = 8
ROW_BLOCK = 1024


def _gating_body(x_ref, w_ref, b_ref, gates_ref, idx_ref):
    logits = (
        jnp.dot(x_ref[...], w_ref[...], preferred_element_type=jnp.float32)
        + b_ref[...]
    )
    iota = jax.lax.broadcasted_iota(jnp.int32, logits.shape, 1)
    work = logits
    mask = jnp.zeros(logits.shape, jnp.bool_)
    idx_cols = []
    for _ in range(TOP_K):
        sel_idx = jnp.argmax(work, axis=1, keepdims=True)
        idx_cols.append(sel_idx)
        sel = iota == sel_idx
        mask = mask | sel
        work = jnp.where(sel, -jnp.inf, work)
    top1 = jnp.max(logits, axis=1, keepdims=True)
    ex = jnp.where(mask, jnp.exp(logits - top1), 0.0)
    gates_ref[...] = ex / jnp.sum(ex, axis=1, keepdims=True)
    idx_ref[...] = jnp.concatenate(idx_cols, axis=1)


def kernel(x, W, b):
    n_tok, d = x.shape
    e = W.shape[1]
    b2 = b.reshape(1, e)
    grid = (n_tok // ROW_BLOCK,)
    gates, idx = pl.pallas_call(
        _gating_body,
        grid=grid,
        in_specs=[
            pl.BlockSpec((ROW_BLOCK, d), lambda i: (i, 0)),
            pl.BlockSpec((d, e), lambda i: (0, 0)),
            pl.BlockSpec((1, e), lambda i: (0, 0)),
        ],
        out_specs=[
            pl.BlockSpec((ROW_BLOCK, e), lambda i: (i, 0)),
            pl.BlockSpec((ROW_BLOCK, TOP_K), lambda i: (i, 0)),
        ],
        out_shape=[
            jax.ShapeDtypeStruct((n_tok, e), jnp.float32),
            jax.ShapeDtypeStruct((n_tok, TOP_K), jnp.int32),
        ],
        compiler_params=pltpu.CompilerParams(
            dimension_semantics=("parallel",),
        ),
    )(x, W, b2)
    return (gates, idx)


# speedup vs baseline: 13.0130x; 1.0633x over previous
import functools

import jax
import jax.numpy as jnp
from jax.experimental import pallas as pl
from jax.experimental.pallas import tpu as pltpu

TOP_K = 8
ROW_BLOCK = 2048


def _gating_body(x_ref, w_ref, b_ref, gates_ref, idx_ref):
    logits = (
        jnp.dot(x_ref[...], w_ref[...], preferred_element_type=jnp.float32)
        + b_ref[...]
    )
    iota = jax.lax.broadcasted_iota(jnp.int32, logits.shape, 1)
    work = logits
    mask = jnp.zeros(logits.shape, jnp.bool_)
    idx_cols = []
    for _ in range(TOP_K):
        sel_idx = jnp.argmax(work, axis=1, keepdims=True)
        idx_cols.append(sel_idx)
        sel = iota == sel_idx
        mask = mask | sel
        work = jnp.where(sel, -jnp.inf, work)
    top1 = jnp.max(logits, axis=1, keepdims=True)
    ex = jnp.where(mask, jnp.exp(logits - top1), 0.0)
    gates_ref[...] = ex / jnp.sum(ex, axis=1, keepdims=True)
    idx_ref[...] = jnp.concatenate(idx_cols, axis=1)


def kernel(x, W, b):
    n_tok, d = x.shape
    e = W.shape[1]
    b2 = b.reshape(1, e)
    grid = (n_tok // ROW_BLOCK,)
    gates, idx = pl.pallas_call(
        _gating_body,
        grid=grid,
        in_specs=[
            pl.BlockSpec((ROW_BLOCK, d), lambda i: (i, 0)),
            pl.BlockSpec((d, e), lambda i: (0, 0)),
            pl.BlockSpec((1, e), lambda i: (0, 0)),
        ],
        out_specs=[
            pl.BlockSpec((ROW_BLOCK, e), lambda i: (i, 0)),
            pl.BlockSpec((ROW_BLOCK, TOP_K), lambda i: (i, 0)),
        ],
        out_shape=[
            jax.ShapeDtypeStruct((n_tok, e), jnp.float32),
            jax.ShapeDtypeStruct((n_tok, TOP_K), jnp.int32),
        ],
        compiler_params=pltpu.CompilerParams(
            dimension_semantics=("parallel",),
        ),
    )(x, W, b2)
    return (gates, idx)
